# baseline (speedup 1.0000x reference)
import jax
import jax.numpy as jnp
from jax import lax
from jax.experimental import pallas as pl
from jax.experimental.pallas import tpu as pltpu
from jax.experimental.pallas import tpu_sc as plsc

_B = 4096
_SL = 50
_L = 128
_NC, _NS = 2, 16
_NW = _NC * _NS
_IPW = _B // _NW
_HALO = 8
_J = 2
_BAND = 64
_NBAND = _IPW // _BAND
_NSTEP = (_SL // _J) * _NBAND
_NBUF = 6


def _sc_roll_body(x_ref, o_ref, *scratch):
    bufs = list(scratch[:_NBUF])
    rsems = list(scratch[_NBUF:2 * _NBUF])
    wsems = list(scratch[2 * _NBUF:])
    wid = lax.axis_index("s") * _NC + lax.axis_index("c")
    s0 = wid * _IPW

    def _coords(g):
        j = (g // _NBAND) * _J
        s = s0 + (g % _NBAND) * _BAND
        h = lax.rem(s + _BAND, _B)
        return j, s, h

    def issue_read(g):
        b = g % _NBUF
        j, s, h = _coords(g)
        return [
            pltpu.async_copy(x_ref.at[pl.ds(j, _J), pl.ds(s, _BAND)],
                             bufs[b].at[:, pl.ds(0, _BAND)], rsems[b]),
            pltpu.async_copy(x_ref.at[pl.ds(j, _J), pl.ds(h, _HALO)],
                             bufs[b].at[:, pl.ds(_BAND, _HALO)], rsems[b]),
        ]

    def issue_write(g):
        b = g % _NBUF
        j, s, _ = _coords(g)
        return [pltpu.async_copy(bufs[b].at[:, pl.ds(1, _BAND)],
                                 o_ref.at[pl.ds(j, _J), pl.ds(s, _BAND)],
                                 wsems[b])]

    depth = _NBUF - 1
    reads, writes = {}, {}
    for g in range(min(depth, _NSTEP)):
        reads[g] = issue_read(g)
    for g in range(_NSTEP):
        nxt = g + depth
        if nxt < _NSTEP:
            if nxt >= _NBUF:
                for h in writes[nxt - _NBUF]:
                    h.wait()
            reads[nxt] = issue_read(nxt)
        for h in reads[g]:
            h.wait()
        writes[g] = issue_write(g)
    for g in range(max(_NSTEP - _NBUF, 0), _NSTEP):
        for h in writes[g]:
            h.wait()


def kernel(x):
    xt = jnp.transpose(x, (1, 0, 2))
    out_t = pl.kernel(
        _sc_roll_body,
        out_type=jax.ShapeDtypeStruct((_SL, _B, _L), jnp.float32),
        mesh=plsc.VectorSubcoreMesh(core_axis_name="c", subcore_axis_name="s"),
        scratch_types=[pltpu.VMEM((_J, _BAND + _HALO, _L), jnp.float32)] * _NBUF
                      + [pltpu.SemaphoreType.DMA] * (2 * _NBUF),
    )(xt)
    return jnp.transpose(out_t, (1, 0, 2))


# speedup vs baseline: 1.0582x; 1.0582x over previous
import jax
import jax.numpy as jnp
from jax import lax
from jax.experimental import pallas as pl
from jax.experimental.pallas import tpu as pltpu
from jax.experimental.pallas import tpu_sc as plsc

_B = 4096
_SL = 50
_L = 128
_NC, _NS = 2, 16
_NW = _NC * _NS
_IPW = _B // _NW
_HALO = 8
_J = 2
_NSTEP = _SL // _J
_NBUF = 3


def _sc_roll_body(x_ref, o_ref, *scratch):
    bufs = list(scratch[:_NBUF])
    rsems = list(scratch[_NBUF:2 * _NBUF])
    wsems = list(scratch[2 * _NBUF:])
    wid = lax.axis_index("s") * _NC + lax.axis_index("c")
    s0 = wid * _IPW
    h0 = lax.rem(s0 + _IPW, _B)

    def issue_read(g):
        b = g % _NBUF
        j = g * _J
        return [
            pltpu.async_copy(x_ref.at[pl.ds(j, _J), pl.ds(s0, _IPW)],
                             bufs[b].at[:, pl.ds(0, _IPW)], rsems[b]),
            pltpu.async_copy(x_ref.at[pl.ds(j, _J), pl.ds(h0, _HALO)],
                             bufs[b].at[:, pl.ds(_IPW, _HALO)], rsems[b]),
        ]

    def issue_write(g):
        b = g % _NBUF
        j = g * _J
        return [pltpu.async_copy(bufs[b].at[:, pl.ds(1, _IPW)],
                                 o_ref.at[pl.ds(j, _J), pl.ds(s0, _IPW)],
                                 wsems[b])]

    depth = _NBUF - 1
    reads, writes = {}, {}
    for g in range(min(depth, _NSTEP)):
        reads[g] = issue_read(g)
    for g in range(_NSTEP):
        for h in reads[g]:
            h.wait()
        writes[g] = issue_write(g)
        nxt = g + depth
        if nxt < _NSTEP:
            if nxt >= _NBUF:
                for h in writes[nxt - _NBUF]:
                    h.wait()
            reads[nxt] = issue_read(nxt)
    for g in range(max(_NSTEP - _NBUF, 0), _NSTEP):
        for h in writes[g]:
            h.wait()


def kernel(x):
    xt = jnp.transpose(x, (1, 0, 2))
    out_t = pl.kernel(
        _sc_roll_body,
        out_type=jax.ShapeDtypeStruct((_SL, _B, _L), jnp.float32),
        mesh=plsc.VectorSubcoreMesh(core_axis_name="c", subcore_axis_name="s"),
        scratch_types=[pltpu.VMEM((_J, _IPW + _HALO, _L), jnp.float32)] * _NBUF
                      + [pltpu.SemaphoreType.DMA] * (2 * _NBUF),
    )(xt)
    return jnp.transpose(out_t, (1, 0, 2))


# speedup vs baseline: 1.0707x; 1.0118x over previous
import jax
import jax.numpy as jnp
from jax import lax
from jax.experimental import pallas as pl
from jax.experimental.pallas import tpu as pltpu
from jax.experimental.pallas import tpu_sc as plsc

_B = 4096
_SL = 50
_L = 128
_NC, _NS = 2, 16
_NW = _NC * _NS
_IPW = _B // _NW
_HALO = 8
_J = 2
_NSTEP = _SL // _J
_NBUF = 3


def _sc_roll_body(x_ref, o_ref, *scratch):
    bufs = list(scratch[:_NBUF])
    rsems = list(scratch[_NBUF:2 * _NBUF])
    wsems = list(scratch[2 * _NBUF:])
    wid = lax.axis_index("s") * _NC + lax.axis_index("c")
    s0 = wid * _IPW
    h0 = lax.rem(s0 + _IPW, _B)

    def issue_read(g):
        b = g % _NBUF
        j = g * _J
        return [
            pltpu.async_copy(x_ref.at[pl.ds(j, _J), pl.ds(s0, _IPW)],
                             bufs[b].at[:, pl.ds(0, _IPW)], rsems[b]),
            pltpu.async_copy(x_ref.at[pl.ds(j, _J), pl.ds(h0, _HALO)],
                             bufs[b].at[:, pl.ds(_IPW, _HALO)], rsems[b]),
        ]

    def issue_write(g):
        b = g % _NBUF
        j = g * _J
        return [pltpu.async_copy(bufs[b].at[:, pl.ds(1, _IPW)],
                                 o_ref.at[pl.ds(j, _J), pl.ds(s0, _IPW)],
                                 wsems[b])]

    depth = _NBUF - 1
    reads, writes = {}, {}
    for g in range(min(depth, _NSTEP)):
        reads[g] = issue_read(g)
    for g in range(_NSTEP):
        nxt = g + depth
        if nxt < _NSTEP:
            if nxt >= _NBUF:
                for h in writes[nxt - _NBUF]:
                    h.wait()
            reads[nxt] = issue_read(nxt)
        for h in reads[g]:
            h.wait()
        writes[g] = issue_write(g)
    for g in range(max(_NSTEP - _NBUF, 0), _NSTEP):
        for h in writes[g]:
            h.wait()


def kernel(x):
    xt = jnp.transpose(x, (1, 0, 2))
    out_t = pl.kernel(
        _sc_roll_body,
        out_type=jax.ShapeDtypeStruct((_SL, _B, _L), jnp.float32),
        mesh=plsc.VectorSubcoreMesh(core_axis_name="c", subcore_axis_name="s"),
        scratch_types=[pltpu.VMEM((_J, _IPW + _HALO, _L), jnp.float32)] * _NBUF
                      + [pltpu.SemaphoreType.DMA] * (2 * _NBUF),
    )(xt)
    return jnp.transpose(out_t, (1, 0, 2))
